# baseline (speedup 1.0000x reference)
import numpy as np
import jax
import jax.numpy as jnp
from jax.experimental import pallas as pl
from jax.experimental.pallas import tpu as pltpu

_DIM = 768
_MAXLEN = 4096
_SCALE = float(np.sqrt(np.float32(_DIM)))


def _pe_table(max_len, dim):
    position = np.arange(max_len, dtype=np.float32)[:, None]
    div_term = np.exp(np.arange(0, dim, 2, dtype=np.float32) * (-np.log(10000.0) / dim))
    pe = np.zeros((max_len, dim), dtype=np.float32)
    pe[:, 0::2] = np.sin(position * div_term)
    pe[:, 1::2] = np.cos(position * div_term)
    return jnp.asarray(pe)


_PE = _pe_table(_MAXLEN, _DIM)

_BL = 512


def _body(t_ref, x_ref, te_ref, pe_ref, o_ref):
    t = t_ref[0]
    te_row = te_ref[pl.ds(t, 1), :]
    bias = pe_ref[...] + te_row * _SCALE
    o_ref[...] = x_ref[...] * _SCALE + bias[None]


def kernel(x, type_emb, t):
    B, L, D = x.shape
    t_arr = jnp.asarray(t, dtype=jnp.int32).reshape((1,))
    pe = _PE[:L]
    grid = (L // _BL, B)
    grid_spec = pltpu.PrefetchScalarGridSpec(
        num_scalar_prefetch=1,
        grid=grid,
        in_specs=[
            pl.BlockSpec((1, _BL, D), lambda i, b, t_ref: (b, i, 0)),
            pl.BlockSpec((2, D), lambda i, b, t_ref: (0, 0)),
            pl.BlockSpec((_BL, D), lambda i, b, t_ref: (i, 0)),
        ],
        out_specs=pl.BlockSpec((1, _BL, D), lambda i, b, t_ref: (b, i, 0)),
    )
    return pl.pallas_call(
        _body,
        grid_spec=grid_spec,
        out_shape=jax.ShapeDtypeStruct(x.shape, x.dtype),
        compiler_params=pltpu.CompilerParams(
            dimension_semantics=("arbitrary", "arbitrary"),
        ),
    )(t_arr, x, type_emb, pe)


# speedup vs baseline: 1.2330x; 1.2330x over previous
import numpy as np
import jax
import jax.numpy as jnp
from jax.experimental import pallas as pl
from jax.experimental.pallas import tpu as pltpu

_DIM = 768
_MAXLEN = 4096
_SCALE = float(np.sqrt(np.float32(_DIM)))


def _pe_table(max_len, dim):
    position = np.arange(max_len, dtype=np.float32)[:, None]
    div_term = np.exp(np.arange(0, dim, 2, dtype=np.float32) * (-np.log(10000.0) / dim))
    pe = np.zeros((max_len, dim), dtype=np.float32)
    pe[:, 0::2] = np.sin(position * div_term)
    pe[:, 1::2] = np.cos(position * div_term)
    return jnp.asarray(pe)


_PE = _pe_table(_MAXLEN, _DIM)

_BL = 512


def _body(t_ref, x_ref, te_ref, pe_ref, o_ref):
    t = t_ref[0]
    te_row = te_ref[pl.ds(t, 1), :]
    bias = pe_ref[...] + te_row * _SCALE
    o_ref[...] = x_ref[...] * _SCALE + bias[None]


def kernel(x, type_emb, t):
    B, L, D = x.shape
    t_arr = jnp.asarray(t, dtype=jnp.int32).reshape((1,))
    pe = _PE[:L]
    grid = (L // _BL,)
    grid_spec = pltpu.PrefetchScalarGridSpec(
        num_scalar_prefetch=1,
        grid=grid,
        in_specs=[
            pl.BlockSpec((B, _BL, D), lambda i, t_ref: (0, i, 0)),
            pl.BlockSpec((2, D), lambda i, t_ref: (0, 0)),
            pl.BlockSpec((_BL, D), lambda i, t_ref: (i, 0)),
        ],
        out_specs=pl.BlockSpec((B, _BL, D), lambda i, t_ref: (0, i, 0)),
    )
    return pl.pallas_call(
        _body,
        grid_spec=grid_spec,
        out_shape=jax.ShapeDtypeStruct(x.shape, x.dtype),
        compiler_params=pltpu.CompilerParams(
            dimension_semantics=("arbitrary",),
        ),
    )(t_arr, x, type_emb, pe)


# speedup vs baseline: 1.3296x; 1.0783x over previous
import numpy as np
import jax
import jax.numpy as jnp
from jax.experimental import pallas as pl
from jax.experimental.pallas import tpu as pltpu

_DIM = 768
_MAXLEN = 4096
_SCALE = float(np.sqrt(np.float32(_DIM)))

_R = 128
_BL = 512
_SUB = _BL // _R


def _pe_tables(max_len, dim):
    position = np.arange(max_len, dtype=np.float64)[:, None]
    div_term = np.exp(np.arange(0, dim, 2, dtype=np.float64) * (-np.log(10000.0) / dim))
    pe = np.zeros((max_len, dim), dtype=np.float64)
    pe[:, 0::2] = np.sin(position * div_term)
    pe[:, 1::2] = np.cos(position * div_term)

    w = np.repeat(div_term, 2)
    k = np.arange(max_len // _R, dtype=np.float64)[:, None]
    ck = np.cos(k * _R * w[None, :])
    sk = np.sin(k * _R * w[None, :])
    sign = np.where(np.arange(dim) % 2 == 0, 1.0, -1.0)
    sk_signed = sk * sign[None, :]

    pe_base = pe[:_R]
    pe_swap = pe_base.reshape(_R, dim // 2, 2)[:, :, ::-1].reshape(_R, dim)

    f32 = lambda a: jnp.asarray(a, dtype=jnp.float32)
    return f32(pe_base), f32(pe_swap), f32(ck), f32(sk_signed)


_PE_BASE, _PE_SWAP, _CK, _SK = _pe_tables(_MAXLEN, _DIM)


def _body(t_ref, x_ref, te_ref, pes_ref, pesw_ref, ck_ref, sk_ref, o_ref):
    t = t_ref[0]
    te_row = te_ref[pl.ds(t, 1), :]
    pes = pes_ref[...]
    pesw = pesw_ref[...]
    ck = ck_ref[0]
    sk = sk_ref[0]
    pe_block = pes[None] * ck[:, None, :] + pesw[None] * sk[:, None, :]
    bias = pe_block.reshape(_BL, _DIM) + te_row * _SCALE
    o_ref[...] = x_ref[...] * _SCALE + bias[None]


def kernel(x, type_emb, t):
    B, L, D = x.shape
    t_arr = jnp.asarray(t, dtype=jnp.int32).reshape((1,))
    grid = (L // _BL,)
    grid_spec = pltpu.PrefetchScalarGridSpec(
        num_scalar_prefetch=1,
        grid=grid,
        in_specs=[
            pl.BlockSpec((B, _BL, D), lambda i, t_ref: (0, i, 0)),
            pl.BlockSpec((2, D), lambda i, t_ref: (0, 0)),
            pl.BlockSpec((_R, D), lambda i, t_ref: (0, 0)),
            pl.BlockSpec((_R, D), lambda i, t_ref: (0, 0)),
            pl.BlockSpec((1, _SUB, D), lambda i, t_ref: (i, 0, 0)),
            pl.BlockSpec((1, _SUB, D), lambda i, t_ref: (i, 0, 0)),
        ],
        out_specs=pl.BlockSpec((B, _BL, D), lambda i, t_ref: (0, i, 0)),
    )
    _call = pl.pallas_call(
        _body,
        grid_spec=grid_spec,
        out_shape=jax.ShapeDtypeStruct(x.shape, x.dtype),
        compiler_params=pltpu.CompilerParams(
            dimension_semantics=("arbitrary",),
        ),
    )
    nb = L // _BL
    ck3 = _CK[: L // _R].reshape(nb, _SUB, D)
    sk3 = _SK[: L // _R].reshape(nb, _SUB, D)
    return _call(t_arr, x, type_emb, _PE_BASE, _PE_SWAP, ck3, sk3)


# speedup vs baseline: 1.4258x; 1.0723x over previous
import numpy as np
import jax
import jax.numpy as jnp
from jax.experimental import pallas as pl
from jax.experimental.pallas import tpu as pltpu

_DIM = 768
_MAXLEN = 4096
_SCALE = float(np.sqrt(np.float32(_DIM)))

_R = 128
_BL = 1024
_SUB = _BL // _R


def _pe_tables(max_len, dim):
    position = np.arange(max_len, dtype=np.float64)[:, None]
    div_term = np.exp(np.arange(0, dim, 2, dtype=np.float64) * (-np.log(10000.0) / dim))
    pe = np.zeros((max_len, dim), dtype=np.float64)
    pe[:, 0::2] = np.sin(position * div_term)
    pe[:, 1::2] = np.cos(position * div_term)

    w = np.repeat(div_term, 2)
    k = np.arange(max_len // _R, dtype=np.float64)[:, None]
    ck = np.cos(k * _R * w[None, :])
    sk = np.sin(k * _R * w[None, :])
    sign = np.where(np.arange(dim) % 2 == 0, 1.0, -1.0)
    sk_signed = sk * sign[None, :]

    pe_base = pe[:_R]
    pe_swap = pe_base.reshape(_R, dim // 2, 2)[:, :, ::-1].reshape(_R, dim)

    f32 = lambda a: jnp.asarray(a, dtype=jnp.float32)
    return f32(pe_base), f32(pe_swap), f32(ck), f32(sk_signed)


_PE_BASE, _PE_SWAP, _CK, _SK = _pe_tables(_MAXLEN, _DIM)


def _body(t_ref, x_ref, te_ref, pes_ref, pesw_ref, ck_ref, sk_ref, o_ref):
    t = t_ref[0]
    te_row = te_ref[pl.ds(t, 1), :]
    pes = pes_ref[...]
    pesw = pesw_ref[...]
    ck = ck_ref[0]
    sk = sk_ref[0]
    pe_block = pes[None] * ck[:, None, :] + pesw[None] * sk[:, None, :]
    bias = pe_block.reshape(_BL, _DIM) + te_row * _SCALE
    o_ref[...] = x_ref[...] * _SCALE + bias[None]


def kernel(x, type_emb, t):
    B, L, D = x.shape
    t_arr = jnp.asarray(t, dtype=jnp.int32).reshape((1,))
    grid = (L // _BL,)
    grid_spec = pltpu.PrefetchScalarGridSpec(
        num_scalar_prefetch=1,
        grid=grid,
        in_specs=[
            pl.BlockSpec((B, _BL, D), lambda i, t_ref: (0, i, 0)),
            pl.BlockSpec((2, D), lambda i, t_ref: (0, 0)),
            pl.BlockSpec((_R, D), lambda i, t_ref: (0, 0)),
            pl.BlockSpec((_R, D), lambda i, t_ref: (0, 0)),
            pl.BlockSpec((1, _SUB, D), lambda i, t_ref: (i, 0, 0)),
            pl.BlockSpec((1, _SUB, D), lambda i, t_ref: (i, 0, 0)),
        ],
        out_specs=pl.BlockSpec((B, _BL, D), lambda i, t_ref: (0, i, 0)),
    )
    _call = pl.pallas_call(
        _body,
        grid_spec=grid_spec,
        out_shape=jax.ShapeDtypeStruct(x.shape, x.dtype),
        compiler_params=pltpu.CompilerParams(
            dimension_semantics=("arbitrary",),
        ),
    )
    nb = L // _BL
    ck3 = _CK[: L // _R].reshape(nb, _SUB, D)
    sk3 = _SK[: L // _R].reshape(nb, _SUB, D)
    return _call(t_arr, x, type_emb, _PE_BASE, _PE_SWAP, ck3, sk3)
